# baseline (speedup 1.0000x reference)
import jax
import jax.numpy as jnp
from jax import lax
from jax.experimental import pallas as pl
from jax.experimental.pallas import tpu as pltpu
from jax.experimental.pallas import tpu_sc as plsc

B = 64
S = 2048
L = 16
SEG = S // L
NC = 2
NS = 16
ROWS_PER_W = B // (NC * NS)
NTOK = 784
BIG = 1 << 20

_I32 = jnp.int32
_F32 = jnp.float32


def _sc_body(tok_hbm, lut_hbm, out_hbm, tok_v, lut_v, acc_v, sh_v, sem):
    wid = lax.axis_index("s") * NC + lax.axis_index("c")

    copies = [
        pltpu.make_async_copy(lut_hbm, lut_v, sem)
    ] + [
        pltpu.make_async_copy(
            tok_hbm.at[wid * ROWS_PER_W + r], tok_v.at[pl.ds(r * S, S)], sem
        )
        for r in range(ROWS_PER_W)
    ]
    for c_ in copies:
        c_.start()
    for c_ in copies:
        c_.wait()

    iota = lax.broadcasted_iota(_I32, (L,), 0)
    pos0 = iota * SEG
    im1 = jnp.maximum(iota - 1, 0)
    lane0 = iota == 0
    zero = jnp.zeros((L,), _I32)
    one = jnp.full((L,), 1, _I32)
    neg1 = jnp.full((L,), -1, _I32)
    c100 = jnp.full((L,), 100, _I32)
    c127 = jnp.full((L,), 127, _I32)
    big = jnp.full((L,), BIG, _I32)

    def shift1(vec, fill):
        sh_v[...] = vec
        sh = plsc.load_gather(sh_v, [im1])
        return jnp.where(lane0, fill, sh)

    def init_row(r):
        base = pos0 + r * S
        tok0 = plsc.load_gather(tok_v, [base])
        lut0 = plsc.load_gather(lut_v, [tok0])
        pcs0 = lut0 & 255
        tm0 = lut0 >= 256
        nm0 = pcs0 < 100
        return dict(
            base=base,
            prev_p=jnp.where(tm0, pos0, neg1),
            prev_v=jnp.where(nm0, tok0, neg1),
            prev_np=jnp.where(nm0, pos0, neg1),
            fp=jnp.where(tm0, pos0, big),
            fv=jnp.where(nm0, tok0, neg1),
            cnt=jnp.where(tm0, one, zero),
            first_pcs=pcs0,
            prev_pcs=pcs0,
        )

    rows = [init_row(r) for r in range(ROWS_PER_W)]
    acc_sd2 = zero
    acc_h = zero
    acc_leap = zero

    nstate = 7
    keys = ("prev_p", "prev_v", "prev_np", "fp", "fv", "cnt", "prev_pcs")

    def body(i, carry):
        accs = list(carry[: 3])
        acc_sd2, acc_h, acc_leap = accs
        pos = pos0 + i
        new_rows = []
        for r in range(ROWS_PER_W):
            st = dict(zip(keys, carry[3 + r * nstate: 3 + (r + 1) * nstate]))
            tok = plsc.load_gather(tok_v, [rows[r]["base"] + i])
            lutv = plsc.load_gather(lut_v, [tok])
            pcs = lutv & 255
            tm = lutv >= 256
            nm = pcs < 100
            cnt = st["cnt"] + lax.shift_right_logical(lutv, 8)
            validt = tm & (st["prev_p"] >= 0)
            d = pos - st["prev_p"]
            acc_sd2 = acc_sd2 + jnp.where(validt, d * d, zero)
            fp = jnp.minimum(st["fp"], jnp.where(tm, pos, big))
            prev_p = jnp.where(tm, pos, st["prev_p"])
            validn = nm & (st["prev_v"] >= 0)
            ivl = tok - st["prev_v"]
            leap = validn & (jnp.abs(ivl) > 12)
            acc_leap = acc_leap + jnp.where(leap, one, zero)
            fv = jnp.where((st["fv"] < 0) & nm, tok, st["fv"])
            prev_v = jnp.where(nm, tok, st["prev_v"])
            prev_np = jnp.where(nm, pos, st["prev_np"])
            harsh = jnp.abs(st["prev_pcs"] - pcs) == 6
            acc_h = acc_h + jnp.where(harsh, one, zero)
            new_rows.append((prev_p, prev_v, prev_np, fp, fv, cnt, pcs))
        out = [acc_sd2, acc_h, acc_leap]
        for t in new_rows:
            out.extend(t)
        return tuple(out)

    init = [acc_sd2, acc_h, acc_leap]
    for st in rows:
        init.extend(st[k] for k in keys)
    carry = lax.fori_loop(1, SEG, body, tuple(init), unroll=7)

    acc_sd2, acc_h, acc_leap = carry[:3]
    acc_n = zero
    acc_sd = zero
    for r in range(ROWS_PER_W):
        st = dict(zip(keys, carry[3 + r * nstate: 3 + (r + 1) * nstate]))
        prev_p, prev_v, prev_np = st["prev_p"], st["prev_v"], st["prev_np"]
        fp, fv, cnt = st["fp"], st["fv"], st["cnt"]
        has_t = fp < big
        acc_n = acc_n + jnp.where(has_t, cnt - one, zero)
        acc_sd = acc_sd + jnp.where(has_t, prev_p - fp, zero)
        pb = shift1(plsc.cummax(prev_p), neg1)
        vb = has_t & (pb >= 0)
        db = fp - pb
        acc_n = acc_n + jnp.where(vb, one, zero)
        acc_sd = acc_sd + jnp.where(vb, db, zero)
        acc_sd2 = acc_sd2 + jnp.where(vb, db * db, zero)
        lastq = jnp.where(prev_np >= 0, prev_np * 128 + prev_v, neg1)
        pq = shift1(plsc.cummax(lastq), neg1)
        vv = (fv >= 0) & (pq >= 0)
        ivb = fv - (pq & c127)
        acc_leap = acc_leap + jnp.where(vv & (jnp.abs(ivb) > 12), one, zero)
        pt = shift1(st["prev_pcs"], c100)
        acc_h = acc_h + jnp.where(jnp.abs(pt - rows[r]["first_pcs"]) == 6,
                                  one, zero)

    acc_v[0] = acc_n
    acc_v[1] = acc_sd
    acc_v[2] = acc_sd2
    acc_v[3] = acc_h
    acc_v[4] = acc_leap
    pltpu.sync_copy(acc_v, out_hbm.at[wid])


_sc_stage = pl.kernel(
    _sc_body,
    out_type=jax.ShapeDtypeStruct((NC * NS, 5, L), _I32),
    mesh=plsc.VectorSubcoreMesh(
        core_axis_name="c", subcore_axis_name="s", num_cores=NC, num_subcores=NS
    ),
    scratch_types=[
        pltpu.VMEM((ROWS_PER_W * S,), _I32),
        pltpu.VMEM((NTOK,), _I32),
        pltpu.VMEM((5, L), _I32),
        pltpu.VMEM((L,), _I32),
        pltpu.SemaphoreType.DMA,
    ],
    compiler_params=pltpu.CompilerParams(needs_layout_passes=False),
)


def _finish_body(p_ref, r_ref, h_ref, v_ref, t_ref):
    x = p_ref[:]
    j = lax.broadcasted_iota(_I32, x.shape, 1)
    z = jnp.zeros_like(x)
    n = jnp.sum(jnp.where(j == 0, x, z))
    sd = jnp.sum(jnp.where(j == 1, x, z))
    sd2 = jnp.sum(jnp.where(j == 2, x, z))
    hh = jnp.sum(jnp.where(j == 3, x, z))
    lp = jnp.sum(jnp.where(j == 4, x, z))
    nf = n.astype(_F32)
    ssd = sd2.astype(_F32) - sd.astype(_F32) * sd.astype(_F32) / nf
    rhythm = jnp.where(
        nf > 0.0, jnp.maximum(ssd, 0.0) / (nf - 1.0) * 0.01, 0.0
    ).astype(_F32)
    harmony = 0.1 * hh.astype(_F32) / float(B * S)
    voice = lp.astype(_F32) / float(B)
    total = rhythm + harmony + 0.5 * voice
    r_ref[0, 0] = rhythm
    h_ref[0, 0] = harmony
    v_ref[0, 0] = voice
    t_ref[0, 0] = total


_finish = pl.pallas_call(
    _finish_body,
    out_shape=[jax.ShapeDtypeStruct((1, 1), _F32)] * 4,
    in_specs=[pl.BlockSpec(memory_space=pltpu.VMEM)],
    out_specs=[pl.BlockSpec(memory_space=pltpu.SMEM)] * 4,
)


@jax.jit
def kernel(generated_tokens):
    t = jnp.arange(NTOK, dtype=jnp.int32)
    lut = jnp.where(t < 128, t % 12, 100) + jnp.where(
        (t >= 256) & (t < 768), 256, 0
    )
    partials = _sc_stage(generated_tokens, lut)
    r, h, v, t = _finish(partials)
    return {
        "rhythm_constraint": r[0, 0],
        "harmony_constraint": h[0, 0],
        "voice_leading_constraint": v[0, 0],
        "total": t[0, 0],
    }


# speedup vs baseline: 1.0057x; 1.0057x over previous
import jax
import jax.numpy as jnp
from jax import lax
from jax.experimental import pallas as pl
from jax.experimental.pallas import tpu as pltpu
from jax.experimental.pallas import tpu_sc as plsc

B = 64
S = 2048
L = 16
SEG = S // L
NC = 2
NS = 16
ROWS_PER_W = B // (NC * NS)
NTOK = 784
BIG = 1 << 20

_I32 = jnp.int32
_F32 = jnp.float32


def _sc_body(tok_hbm, lut_hbm, out_hbm, tok_v, lut_v, acc_v, sh_v, sem):
    wid = lax.axis_index("s") * NC + lax.axis_index("c")

    copies = [
        pltpu.make_async_copy(lut_hbm, lut_v, sem)
    ] + [
        pltpu.make_async_copy(
            tok_hbm.at[wid * ROWS_PER_W + r], tok_v.at[pl.ds(r * S, S)], sem
        )
        for r in range(ROWS_PER_W)
    ]
    for c_ in copies:
        c_.start()
    for c_ in copies:
        c_.wait()

    iota = lax.broadcasted_iota(_I32, (L,), 0)
    pos0 = iota * SEG
    im1 = jnp.maximum(iota - 1, 0)
    lane0 = iota == 0
    zero = jnp.zeros((L,), _I32)
    one = jnp.full((L,), 1, _I32)
    neg1 = jnp.full((L,), -1, _I32)
    c100 = jnp.full((L,), 100, _I32)
    c127 = jnp.full((L,), 127, _I32)
    big = jnp.full((L,), BIG, _I32)

    def shift1(vec, fill):
        sh_v[...] = vec
        sh = plsc.load_gather(sh_v, [im1])
        return jnp.where(lane0, fill, sh)

    def init_row(r):
        base = pos0 + r * S
        tok0 = plsc.load_gather(tok_v, [base])
        lut0 = plsc.load_gather(lut_v, [tok0])
        pcs0 = lut0 & 255
        tm0 = lut0 >= 256
        nm0 = pcs0 < 100
        return dict(
            base=base,
            prev_p=jnp.where(tm0, pos0, neg1),
            prev_v=jnp.where(nm0, tok0, neg1),
            prev_np=jnp.where(nm0, pos0, neg1),
            fp=jnp.where(tm0, pos0, big),
            fv=jnp.where(nm0, tok0, neg1),
            cnt=jnp.where(tm0, one, zero),
            first_pcs=pcs0,
            prev_pcs=pcs0,
        )

    rows = [init_row(r) for r in range(ROWS_PER_W)]
    acc_sd2 = zero
    acc_h = zero
    acc_leap = zero

    nstate = 7
    keys = ("prev_p", "prev_v", "prev_np", "fp", "fv", "cnt", "prev_pcs")

    def body(i, carry):
        accs = list(carry[: 3])
        acc_sd2, acc_h, acc_leap = accs
        pos = pos0 + i
        new_rows = []
        for r in range(ROWS_PER_W):
            st = dict(zip(keys, carry[3 + r * nstate: 3 + (r + 1) * nstate]))
            tok = plsc.load_gather(tok_v, [rows[r]["base"] + i])
            lutv = plsc.load_gather(lut_v, [tok])
            pcs = lutv & 255
            tm = lutv >= 256
            nm = pcs < 100
            cnt = st["cnt"] + lax.shift_right_logical(lutv, 8)
            validt = tm & (st["prev_p"] >= 0)
            d = pos - st["prev_p"]
            acc_sd2 = acc_sd2 + jnp.where(validt, d * d, zero)
            fp = jnp.minimum(st["fp"], jnp.where(tm, pos, big))
            prev_p = jnp.where(tm, pos, st["prev_p"])
            validn = nm & (st["prev_v"] >= 0)
            ivl = tok - st["prev_v"]
            leap = validn & (jnp.abs(ivl) > 12)
            acc_leap = acc_leap + jnp.where(leap, one, zero)
            fv = jnp.where((st["fv"] < 0) & nm, tok, st["fv"])
            prev_v = jnp.where(nm, tok, st["prev_v"])
            prev_np = jnp.where(nm, pos, st["prev_np"])
            harsh = jnp.abs(st["prev_pcs"] - pcs) == 6
            acc_h = acc_h + jnp.where(harsh, one, zero)
            new_rows.append((prev_p, prev_v, prev_np, fp, fv, cnt, pcs))
        out = [acc_sd2, acc_h, acc_leap]
        for t in new_rows:
            out.extend(t)
        return tuple(out)

    init = [acc_sd2, acc_h, acc_leap]
    for st in rows:
        init.extend(st[k] for k in keys)
    carry = lax.fori_loop(1, SEG, body, tuple(init), unroll=9)

    acc_sd2, acc_h, acc_leap = carry[:3]
    acc_n = zero
    acc_sd = zero
    for r in range(ROWS_PER_W):
        st = dict(zip(keys, carry[3 + r * nstate: 3 + (r + 1) * nstate]))
        prev_p, prev_v, prev_np = st["prev_p"], st["prev_v"], st["prev_np"]
        fp, fv, cnt = st["fp"], st["fv"], st["cnt"]
        has_t = fp < big
        acc_n = acc_n + jnp.where(has_t, cnt - one, zero)
        acc_sd = acc_sd + jnp.where(has_t, prev_p - fp, zero)
        pb = shift1(plsc.cummax(prev_p), neg1)
        vb = has_t & (pb >= 0)
        db = fp - pb
        acc_n = acc_n + jnp.where(vb, one, zero)
        acc_sd = acc_sd + jnp.where(vb, db, zero)
        acc_sd2 = acc_sd2 + jnp.where(vb, db * db, zero)
        lastq = jnp.where(prev_np >= 0, prev_np * 128 + prev_v, neg1)
        pq = shift1(plsc.cummax(lastq), neg1)
        vv = (fv >= 0) & (pq >= 0)
        ivb = fv - (pq & c127)
        acc_leap = acc_leap + jnp.where(vv & (jnp.abs(ivb) > 12), one, zero)
        pt = shift1(st["prev_pcs"], c100)
        acc_h = acc_h + jnp.where(jnp.abs(pt - rows[r]["first_pcs"]) == 6,
                                  one, zero)

    acc_v[0] = acc_n
    acc_v[1] = acc_sd
    acc_v[2] = acc_sd2
    acc_v[3] = acc_h
    acc_v[4] = acc_leap
    pltpu.sync_copy(acc_v, out_hbm.at[wid])


_sc_stage = pl.kernel(
    _sc_body,
    out_type=jax.ShapeDtypeStruct((NC * NS, 5, L), _I32),
    mesh=plsc.VectorSubcoreMesh(
        core_axis_name="c", subcore_axis_name="s", num_cores=NC, num_subcores=NS
    ),
    scratch_types=[
        pltpu.VMEM((ROWS_PER_W * S,), _I32),
        pltpu.VMEM((NTOK,), _I32),
        pltpu.VMEM((5, L), _I32),
        pltpu.VMEM((L,), _I32),
        pltpu.SemaphoreType.DMA,
    ],
    compiler_params=pltpu.CompilerParams(needs_layout_passes=False),
)


def _finish_body(p_ref, r_ref, h_ref, v_ref, t_ref):
    x = p_ref[:]
    j = lax.broadcasted_iota(_I32, x.shape, 1)
    z = jnp.zeros_like(x)
    n = jnp.sum(jnp.where(j == 0, x, z))
    sd = jnp.sum(jnp.where(j == 1, x, z))
    sd2 = jnp.sum(jnp.where(j == 2, x, z))
    hh = jnp.sum(jnp.where(j == 3, x, z))
    lp = jnp.sum(jnp.where(j == 4, x, z))
    nf = n.astype(_F32)
    ssd = sd2.astype(_F32) - sd.astype(_F32) * sd.astype(_F32) / nf
    rhythm = jnp.where(
        nf > 0.0, jnp.maximum(ssd, 0.0) / (nf - 1.0) * 0.01, 0.0
    ).astype(_F32)
    harmony = 0.1 * hh.astype(_F32) / float(B * S)
    voice = lp.astype(_F32) / float(B)
    total = rhythm + harmony + 0.5 * voice
    r_ref[0, 0] = rhythm
    h_ref[0, 0] = harmony
    v_ref[0, 0] = voice
    t_ref[0, 0] = total


_finish = pl.pallas_call(
    _finish_body,
    out_shape=[jax.ShapeDtypeStruct((1, 1), _F32)] * 4,
    in_specs=[pl.BlockSpec(memory_space=pltpu.VMEM)],
    out_specs=[pl.BlockSpec(memory_space=pltpu.SMEM)] * 4,
)


@jax.jit
def kernel(generated_tokens):
    t = jnp.arange(NTOK, dtype=jnp.int32)
    lut = jnp.where(t < 128, t % 12, 100) + jnp.where(
        (t >= 256) & (t < 768), 256, 0
    )
    partials = _sc_stage(generated_tokens, lut)
    r, h, v, t = _finish(partials)
    return {
        "rhythm_constraint": r[0, 0],
        "harmony_constraint": h[0, 0],
        "voice_leading_constraint": v[0, 0],
        "total": t[0, 0],
    }
